# baseline (speedup 1.0000x reference)
import numpy as np
import jax
import jax.numpy as jnp
from jax import lax
from jax.experimental import pallas as pl
from jax.experimental.pallas import tpu as pltpu
from jax.experimental.pallas import tpu_sc as plsc

_N = 4096
_P = _N * (_N - 1) // 2
_C = 4032
_CPW = 65
_L = 16
_PAD = _C + _L
_DPAD = 3 * _C + 3 * _L
_CUT2 = 25.0
_MAGIC = 12582912.0


def _q(v):
    b = plsc.bitcast(v, jnp.int32)
    b = (b + jnp.int32(32767) + ((b >> 16) & 1)) & jnp.int32(-65536)
    return plsc.bitcast(b, jnp.float32)


def _sc_body(prm_hbm, px_hbm, py_hbm, pz_hbm,
             nbr_hbm, dlx_hbm, dly_hbm, dlz_hbm, dst_hbm,
             prm_v, px_v, py_v, pz_v,
             nbr0_a, nbr1_a, dlx_a, dly_a, dlz_a, dst_a,
             nbr0_b, nbr1_b, dlx_b, dly_b, dlz_b, dst_b,
             sem_a, sem_b):
    cid = lax.axis_index("c")
    sid = lax.axis_index("s")
    wid = sid * 2 + cid

    pltpu.sync_copy(prm_hbm, prm_v)
    pltpu.sync_copy(px_hbm, px_v)
    pltpu.sync_copy(py_hbm, py_v)
    pltpu.sync_copy(pz_hbm, pz_v)

    pv = prm_v[pl.ds(0, _L)]
    icx = jnp.full((_L,), pv[0])
    icy = jnp.full((_L,), pv[1])
    icz = jnp.full((_L,), pv[2])
    cx = jnp.full((_L,), pv[3])
    cy = jnp.full((_L,), pv[4])
    cz = jnp.full((_L,), pv[5])
    iota = lax.iota(jnp.int32, _L)
    iota3 = iota * 3

    pstart = wid * (_CPW * _C)

    def bs_body(_, c):
        lo, hi = c
        mid = (lo + hi + 1) >> 1
        off = mid * (_N - 1) - ((mid * (mid - 1)) // 2)
        take = off <= pstart
        return (jnp.where(take, mid, lo), jnp.where(take, hi, mid - 1))

    i0, _ = lax.fori_loop(0, 13, bs_body, (jnp.int32(0), jnp.int32(_N - 1)))
    j0 = i0 + 1 + (pstart - (i0 * (_N - 1) - ((i0 * (i0 - 1)) // 2)))

    def fire(t, bufs, sem):
        nbr0_v, nbr1_v, dlx_v, dly_v, dlz_v, dst_v = bufs
        p0 = pstart + t * _C
        pltpu.async_copy(dst_v.at[pl.ds(0, _C)],
                         dst_hbm.at[pl.ds(p0, _C)], sem)
        pltpu.async_copy(nbr0_v.at[pl.ds(0, _C)],
                         nbr_hbm.at[pl.ds(p0, _C)], sem)
        pltpu.async_copy(nbr1_v.at[pl.ds(0, _C)],
                         nbr_hbm.at[pl.ds(_P + p0, _C)], sem)
        pltpu.async_copy(dlx_v.at[pl.ds(0, _C)],
                         dlx_hbm.at[pl.ds(p0, _C)], sem)
        pltpu.async_copy(dly_v.at[pl.ds(0, _C)],
                         dly_hbm.at[pl.ds(p0, _C)], sem)
        pltpu.async_copy(dlz_v.at[pl.ds(0, _C)],
                         dlz_hbm.at[pl.ds(p0, _C)], sem)

    def drain(bufs, sem):
        nbr0_v, nbr1_v, dlx_v, dly_v, dlz_v, dst_v = bufs
        pltpu.make_async_copy(dst_v.at[pl.ds(0, _C)],
                              dst_hbm.at[pl.ds(0, _C)], sem).wait()
        pltpu.make_async_copy(nbr0_v.at[pl.ds(0, _C)],
                              nbr_hbm.at[pl.ds(0, _C)], sem).wait()
        pltpu.make_async_copy(nbr1_v.at[pl.ds(0, _C)],
                              nbr_hbm.at[pl.ds(0, _C)], sem).wait()
        pltpu.make_async_copy(dlx_v.at[pl.ds(0, _C)],
                              dlx_hbm.at[pl.ds(0, _C)], sem).wait()
        pltpu.make_async_copy(dly_v.at[pl.ds(0, _C)],
                              dly_hbm.at[pl.ds(0, _C)], sem).wait()
        pltpu.make_async_copy(dlz_v.at[pl.ds(0, _C)],
                              dlz_hbm.at[pl.ds(0, _C)], sem).wait()

    def compute_chunk(bufs, carry):
        nbr0_v, nbr1_v, dlx_v, dly_v, dlz_v, dst_v = bufs
        i_in, j_in = carry

        def seg_cond(c):
            return c[2] < _C

        def seg_body(c):
            i, j, q = c
            row_rem = _N - j
            seg = jnp.minimum(row_rem, _C - q)
            xi = jnp.full((_L,), px_v[pl.ds(i, _L)][0])
            yi = jnp.full((_L,), py_v[pl.ds(i, _L)][0])
            zi = jnp.full((_L,), pz_v[pl.ds(i, _L)][0])
            ivec = jnp.full((_L,), i)
            nv = (seg + _L - 1) // _L

            def vec_body(v, carry2):
                o = q + v * _L
                jv = j + v * _L
                xj = px_v[pl.ds(jv, _L)]
                yj = py_v[pl.ds(jv, _L)]
                zj = pz_v[pl.ds(jv, _L)]
                fx = _q(xi - xj) * icx
                fy = _q(yi - yj) * icy
                fz = _q(zi - zj) * icz
                rx = (fx + _MAGIC) - _MAGIC
                ry = (fy + _MAGIC) - _MAGIC
                rz = (fz + _MAGIC) - _MAGIC
                dx = _q(fx - rx) * cx
                dy = _q(fy - ry) * cy
                dz = _q(fz - rz) * cz
                d2 = dx * dx + dy * dy + dz * dz
                bits = plsc.bitcast(d2, jnp.int32)
                y = plsc.bitcast(jnp.int32(0x5F3759DF) - (bits >> 1),
                                 jnp.float32)
                xh = 0.5 * d2
                y = y * (1.5 - xh * y * y)
                y = y * (1.5 - xh * y * y)
                y = y * (1.5 - xh * y * y)
                dist = jnp.where(d2 > 0.0, d2 * y, 0.0)
                mask = d2 < _CUT2
                sl = pl.ds(o, _L)
                dst_v[sl] = jnp.where(mask, dist, 0.0)
                nbr0_v[sl] = jnp.where(mask, ivec, -1)
                nbr1_v[sl] = jnp.where(mask, jv + iota, -1)
                dlx_v[sl] = jnp.where(mask, dx, 0.0)
                dly_v[sl] = jnp.where(mask, dy, 0.0)
                dlz_v[sl] = jnp.where(mask, dz, 0.0)
                return carry2

            lax.fori_loop(0, nv, vec_body, 0)
            done_row = seg == row_rem
            return (jnp.where(done_row, i + 1, i),
                    jnp.where(done_row, i + 2, j + seg),
                    q + seg)

        i_out, j_out, _ = lax.while_loop(
            seg_cond, seg_body, (i_in, j_in, jnp.int32(0)))
        return (i_out, j_out)

    bufs_a = (nbr0_a, nbr1_a, dlx_a, dly_a, dlz_a, dst_a)
    bufs_b = (nbr0_b, nbr1_b, dlx_b, dly_b, dlz_b, dst_b)

    def pair_body(tt, carry):
        @pl.when(tt > 0)
        def _():
            drain(bufs_a, sem_a)
        carry = compute_chunk(bufs_a, carry)
        fire(2 * tt, bufs_a, sem_a)

        @pl.when(tt > 0)
        def _():
            drain(bufs_b, sem_b)
        carry = compute_chunk(bufs_b, carry)
        fire(2 * tt + 1, bufs_b, sem_b)
        return carry

    carry = lax.fori_loop(0, _CPW // 2, pair_body, (i0, j0))
    drain(bufs_a, sem_a)
    carry = compute_chunk(bufs_a, carry)
    fire(_CPW - 1, bufs_a, sem_a)
    drain(bufs_b, sem_b)
    drain(bufs_a, sem_a)


def kernel(positions, cell):
    pos_t = positions.T.astype(jnp.float32)
    pad = jnp.zeros((_L,), jnp.float32)
    px = jnp.concatenate([pos_t[0], pad])
    py = jnp.concatenate([pos_t[1], pad])
    pz = jnp.concatenate([pos_t[2], pad])
    inv_cell = jnp.linalg.inv(cell)
    prm = jnp.stack([
        inv_cell[0, 0], inv_cell[1, 1], inv_cell[2, 2],
        cell[0, 0], cell[1, 1], cell[2, 2],
    ]).astype(jnp.bfloat16).astype(jnp.float32)
    prm = jnp.concatenate([prm, jnp.zeros((10,), jnp.float32)])

    mesh = plsc.VectorSubcoreMesh(core_axis_name="c", subcore_axis_name="s")
    run = pl.kernel(
        _sc_body,
        out_type=[
            jax.ShapeDtypeStruct((2 * _P,), jnp.int32),
            jax.ShapeDtypeStruct((_P,), jnp.float32),
            jax.ShapeDtypeStruct((_P,), jnp.float32),
            jax.ShapeDtypeStruct((_P,), jnp.float32),
            jax.ShapeDtypeStruct((_P,), jnp.float32),
        ],
        mesh=mesh,
        compiler_params=pltpu.CompilerParams(needs_layout_passes=False,
                                             use_tc_tiling_on_sc=False),
        scratch_types=[
            pltpu.VMEM((16,), jnp.float32),
            pltpu.VMEM((_N + _L,), jnp.float32),
            pltpu.VMEM((_N + _L,), jnp.float32),
            pltpu.VMEM((_N + _L,), jnp.float32),
            pltpu.VMEM((_PAD,), jnp.int32),
            pltpu.VMEM((_PAD,), jnp.int32),
            pltpu.VMEM((_PAD,), jnp.float32),
            pltpu.VMEM((_PAD,), jnp.float32),
            pltpu.VMEM((_PAD,), jnp.float32),
            pltpu.VMEM((_PAD,), jnp.float32),
            pltpu.VMEM((_PAD,), jnp.int32),
            pltpu.VMEM((_PAD,), jnp.int32),
            pltpu.VMEM((_PAD,), jnp.float32),
            pltpu.VMEM((_PAD,), jnp.float32),
            pltpu.VMEM((_PAD,), jnp.float32),
            pltpu.VMEM((_PAD,), jnp.float32),
            pltpu.SemaphoreType.DMA,
            pltpu.SemaphoreType.DMA,
        ],
    )
    nbr_flat, dlx, dly, dlz, dst = run(prm, px, py, pz)
    deltas = jnp.stack([dlx, dly, dlz], axis=0).T
    return nbr_flat.reshape(2, _P), deltas, dst


# speedup vs baseline: 1.0024x; 1.0024x over previous
import jax
import jax.numpy as jnp
from jax import lax
from jax.experimental import pallas as pl
from jax.experimental.pallas import tpu as pltpu
from jax.experimental.pallas import tpu_sc as plsc

_N = 4096
_P = _N * (_N - 1) // 2
_C = 4032
_CPW = 65
_L = 16
_PAD = _C + _L
_CUT2 = 25.0
_MAGIC = 12582912.0


def _q(v):
    b = plsc.bitcast(v, jnp.int32)
    b = (b + jnp.int32(32767) + ((b >> 16) & 1)) & jnp.int32(-65536)
    return plsc.bitcast(b, jnp.float32)


def _sc_body(prm_hbm, px_hbm, py_hbm, pz_hbm,
             nbr_hbm, dlx_hbm, dly_hbm, dlz_hbm, dst_hbm,
             prm_v, px_v, py_v, pz_v,
             nbr0_a, nbr1_a, dlx_a, dly_a, dlz_a, dst_a,
             nbr0_b, nbr1_b, dlx_b, dly_b, dlz_b, dst_b,
             sem_a, sem_b):
    cid = lax.axis_index("c")
    sid = lax.axis_index("s")
    wid = sid * 2 + cid

    pltpu.sync_copy(prm_hbm, prm_v)
    pltpu.sync_copy(px_hbm, px_v)
    pltpu.sync_copy(py_hbm, py_v)
    pltpu.sync_copy(pz_hbm, pz_v)

    pv = prm_v[pl.ds(0, _L)]
    icx = jnp.full((_L,), pv[0])
    icy = jnp.full((_L,), pv[1])
    icz = jnp.full((_L,), pv[2])
    cx = jnp.full((_L,), pv[3])
    cy = jnp.full((_L,), pv[4])
    cz = jnp.full((_L,), pv[5])
    iota = lax.iota(jnp.int32, _L)
    iota3 = iota * 3

    pstart = wid * (_CPW * _C)

    def bs_body(_, c):
        lo, hi = c
        mid = (lo + hi + 1) >> 1
        off = mid * (_N - 1) - ((mid * (mid - 1)) // 2)
        take = off <= pstart
        return (jnp.where(take, mid, lo), jnp.where(take, hi, mid - 1))

    i0, _ = lax.fori_loop(0, 13, bs_body, (jnp.int32(0), jnp.int32(_N - 1)))
    j0 = i0 + 1 + (pstart - (i0 * (_N - 1) - ((i0 * (i0 - 1)) // 2)))

    def fire(t, bufs, sem):
        nbr0_v, nbr1_v, dlx_v, dly_v, dlz_v, dst_v = bufs
        p0 = pstart + t * _C
        pltpu.async_copy(dst_v.at[pl.ds(0, _C)],
                         dst_hbm.at[pl.ds(p0, _C)], sem)
        pltpu.async_copy(nbr0_v.at[pl.ds(0, _C)],
                         nbr_hbm.at[pl.ds(p0, _C)], sem)
        pltpu.async_copy(nbr1_v.at[pl.ds(0, _C)],
                         nbr_hbm.at[pl.ds(_P + p0, _C)], sem)
        pltpu.async_copy(dlx_v.at[pl.ds(0, _C)],
                         dlx_hbm.at[pl.ds(p0, _C)], sem)
        pltpu.async_copy(dly_v.at[pl.ds(0, _C)],
                         dly_hbm.at[pl.ds(p0, _C)], sem)
        pltpu.async_copy(dlz_v.at[pl.ds(0, _C)],
                         dlz_hbm.at[pl.ds(p0, _C)], sem)

    def drain(bufs, sem):
        nbr0_v, nbr1_v, dlx_v, dly_v, dlz_v, dst_v = bufs
        pltpu.make_async_copy(dst_v.at[pl.ds(0, _C)],
                              dst_hbm.at[pl.ds(0, _C)], sem).wait()
        pltpu.make_async_copy(nbr0_v.at[pl.ds(0, _C)],
                              nbr_hbm.at[pl.ds(0, _C)], sem).wait()
        pltpu.make_async_copy(nbr1_v.at[pl.ds(0, _C)],
                              nbr_hbm.at[pl.ds(0, _C)], sem).wait()
        pltpu.make_async_copy(dlx_v.at[pl.ds(0, _C)],
                              dlx_hbm.at[pl.ds(0, _C)], sem).wait()
        pltpu.make_async_copy(dly_v.at[pl.ds(0, _C)],
                              dly_hbm.at[pl.ds(0, _C)], sem).wait()
        pltpu.make_async_copy(dlz_v.at[pl.ds(0, _C)],
                              dlz_hbm.at[pl.ds(0, _C)], sem).wait()

    def compute_chunk(bufs, carry):
        nbr0_v, nbr1_v, dlx_v, dly_v, dlz_v, dst_v = bufs
        i_in, j_in = carry

        def seg_cond(c):
            return c[2] < _C

        def seg_body(c):
            i, j, q = c
            row_rem = _N - j
            seg = jnp.minimum(row_rem, _C - q)
            xi = jnp.full((_L,), px_v[pl.ds(i, _L)][0])
            yi = jnp.full((_L,), py_v[pl.ds(i, _L)][0])
            zi = jnp.full((_L,), pz_v[pl.ds(i, _L)][0])
            ivec = jnp.full((_L,), i)
            nv = (seg + _L - 1) // _L

            def vec_body(v, carry2):
                o = q + v * _L
                jv = j + v * _L
                xj = px_v[pl.ds(jv, _L)]
                yj = py_v[pl.ds(jv, _L)]
                zj = pz_v[pl.ds(jv, _L)]
                fx = _q(xi - xj) * icx
                fy = _q(yi - yj) * icy
                fz = _q(zi - zj) * icz
                rx = (fx + _MAGIC) - _MAGIC
                ry = (fy + _MAGIC) - _MAGIC
                rz = (fz + _MAGIC) - _MAGIC
                dx = _q(fx - rx) * cx
                dy = _q(fy - ry) * cy
                dz = _q(fz - rz) * cz
                d2 = dx * dx + dy * dy + dz * dz
                bits = plsc.bitcast(d2, jnp.int32)
                y = plsc.bitcast(jnp.int32(0x5F3759DF) - (bits >> 1),
                                 jnp.float32)
                xh = 0.5 * d2
                y = y * (1.5 - xh * y * y)
                y = y * (1.5 - xh * y * y)
                y = y * (1.5 - xh * y * y)
                dist = jnp.where(d2 > 0.0, d2 * y, 0.0)
                mask = d2 < _CUT2
                sl = pl.ds(o, _L)
                dst_v[sl] = jnp.where(mask, dist, 0.0)
                nbr0_v[sl] = jnp.where(mask, ivec, -1)
                nbr1_v[sl] = jnp.where(mask, jv + iota, -1)
                dlx_v[sl] = jnp.where(mask, dx, 0.0)
                dly_v[sl] = jnp.where(mask, dy, 0.0)
                dlz_v[sl] = jnp.where(mask, dz, 0.0)
                return carry2

            lax.fori_loop(0, nv, vec_body, 0)
            done_row = seg == row_rem
            return (jnp.where(done_row, i + 1, i),
                    jnp.where(done_row, i + 2, j + seg),
                    q + seg)

        i_out, j_out, _ = lax.while_loop(
            seg_cond, seg_body, (i_in, j_in, jnp.int32(0)))
        return (i_out, j_out)

    bufs_a = (nbr0_a, nbr1_a, dlx_a, dly_a, dlz_a, dst_a)
    bufs_b = (nbr0_b, nbr1_b, dlx_b, dly_b, dlz_b, dst_b)

    def pair_body(tt, carry):
        @pl.when(tt > 0)
        def _():
            drain(bufs_a, sem_a)
        carry = compute_chunk(bufs_a, carry)
        fire(2 * tt, bufs_a, sem_a)

        @pl.when(tt > 0)
        def _():
            drain(bufs_b, sem_b)
        carry = compute_chunk(bufs_b, carry)
        fire(2 * tt + 1, bufs_b, sem_b)
        return carry

    carry = lax.fori_loop(0, _CPW // 2, pair_body, (i0, j0))
    drain(bufs_a, sem_a)
    carry = compute_chunk(bufs_a, carry)
    fire(_CPW - 1, bufs_a, sem_a)
    drain(bufs_b, sem_b)
    drain(bufs_a, sem_a)


def kernel(positions, cell):
    pos_t = positions.T.astype(jnp.float32)
    pad = jnp.zeros((_L,), jnp.float32)
    px = jnp.concatenate([pos_t[0], pad])
    py = jnp.concatenate([pos_t[1], pad])
    pz = jnp.concatenate([pos_t[2], pad])
    inv_cell = jnp.linalg.inv(cell)
    prm = jnp.stack([
        inv_cell[0, 0], inv_cell[1, 1], inv_cell[2, 2],
        cell[0, 0], cell[1, 1], cell[2, 2],
    ]).astype(jnp.bfloat16).astype(jnp.float32)
    prm = jnp.concatenate([prm, jnp.zeros((10,), jnp.float32)])

    mesh = plsc.VectorSubcoreMesh(core_axis_name="c", subcore_axis_name="s")
    run = pl.kernel(
        _sc_body,
        out_type=[
            jax.ShapeDtypeStruct((2 * _P,), jnp.int32),
            jax.ShapeDtypeStruct((_P,), jnp.float32),
            jax.ShapeDtypeStruct((_P,), jnp.float32),
            jax.ShapeDtypeStruct((_P,), jnp.float32),
            jax.ShapeDtypeStruct((_P,), jnp.float32),
        ],
        mesh=mesh,
        compiler_params=pltpu.CompilerParams(needs_layout_passes=False,
                                             use_tc_tiling_on_sc=False),
        scratch_types=[
            pltpu.VMEM((16,), jnp.float32),
            pltpu.VMEM((_N + _L,), jnp.float32),
            pltpu.VMEM((_N + _L,), jnp.float32),
            pltpu.VMEM((_N + _L,), jnp.float32),
            pltpu.VMEM((_PAD,), jnp.int32),
            pltpu.VMEM((_PAD,), jnp.int32),
            pltpu.VMEM((_PAD,), jnp.float32),
            pltpu.VMEM((_PAD,), jnp.float32),
            pltpu.VMEM((_PAD,), jnp.float32),
            pltpu.VMEM((_PAD,), jnp.float32),
            pltpu.VMEM((_PAD,), jnp.int32),
            pltpu.VMEM((_PAD,), jnp.int32),
            pltpu.VMEM((_PAD,), jnp.float32),
            pltpu.VMEM((_PAD,), jnp.float32),
            pltpu.VMEM((_PAD,), jnp.float32),
            pltpu.VMEM((_PAD,), jnp.float32),
            pltpu.SemaphoreType.DMA,
            pltpu.SemaphoreType.DMA,
        ],
    )
    nbr_flat, dlx, dly, dlz, dst = run(prm, px, py, pz)
    deltas = jnp.stack([dlx, dly, dlz], axis=-1)
    return nbr_flat.reshape(2, _P), deltas, dst
